# baseline (speedup 1.0000x reference)
import functools

import jax
import jax.numpy as jnp
from jax import lax
from jax.experimental import pallas as pl
from jax.experimental.pallas import tpu as pltpu, tpu_sc as plsc

N = 100000
E = 1600000
G = 128
VOCAB = 8018

NC = 2
NS = 16
NW = NC * NS
NPT = N // NW
EPT = E // NW

_MESH = plsc.VectorSubcoreMesh(core_axis_name="c", subcore_axis_name="s")
_SC_PARAMS = pltpu.CompilerParams(
    needs_layout_passes=False, use_tc_tiling_on_sc=False)

SPILL_CAP = E + 4096


def _rsqrt_pos(x):
    i = plsc.bitcast(x, jnp.int32)
    y = plsc.bitcast(jnp.int32(0x5F3759DF) - (i >> 1), jnp.float32)
    for _ in range(3):
        y = y * (1.5 - 0.5 * x * y * y)
    return y


@functools.partial(
    pl.kernel, mesh=_MESH, compiler_params=_SC_PARAMS,
    out_type=(
        jax.ShapeDtypeStruct((N, 16), jnp.float32),
        jax.ShapeDtypeStruct((N, 16), jnp.float32),
        jax.ShapeDtypeStruct((2, N), jnp.float32),
        jax.ShapeDtypeStruct((2, N), jnp.float32),
        jax.ShapeDtypeStruct((2, N), jnp.float32),
        jax.ShapeDtypeStruct((2, N), jnp.float32),
    ),
    scratch_types=[
        pltpu.VMEM((1000,), jnp.int32),
        pltpu.VMEM((1000, 16), jnp.float32),
        pltpu.VMEM((2000,), jnp.int32),
        pltpu.VMEM((2000,), jnp.float32),
        pltpu.VMEM((2000,), jnp.float32),
        pltpu.VMEM_SHARED((N,), jnp.float32),
        pltpu.VMEM_SHARED((N,), jnp.float32),
        pltpu.VMEM_SHARED((N,), jnp.float32),
        pltpu.VMEM_SHARED((N,), jnp.float32),
        pltpu.SemaphoreType.DMA,
    ],
)
def _sc_prep(tok1, tok2, embed, src1, dst1, src2, dst2,
             h1o, h2o, do1o, di1o, do2o, di2o,
             tid_v, rows_v, eid_v, ones_v, zb_v,
             p_do1, p_di1, p_do2, p_di2, sem):
    cid = lax.axis_index("c")
    sid = lax.axis_index("s")
    wid = sid * NC + cid

    def z16(i):
        zb_v[pl.ds(i * 16, 16)] = jnp.zeros((16,), jnp.float32)
        ones_v[pl.ds(i * 16, 16)] = jnp.ones((16,), jnp.float32)
    pl.loop(0, 2000 // 16)(z16)

    def zblk(b):
        blk = sid + b * NS
        off = pl.multiple_of(blk * 2000, 8)
        for p in (p_do1, p_di1, p_do2, p_di2):
            pltpu.sync_copy(zb_v, p.at[pl.ds(off, 2000)])
    pl.loop(0, 50 // NS + 1)(lambda b: pl.when(sid + b * NS < 50)(
        lambda: zblk(b)))
    plsc.subcore_barrier()

    def deg(b, idx_hbm, plane):
        off = pl.multiple_of(wid * EPT + b * 2000, 8)
        pltpu.sync_copy(idx_hbm.at[pl.ds(off, 2000)], eid_v)
        pltpu.sync_copy(ones_v, plane.at[eid_v], add=True)
    pl.loop(0, EPT // 2000)(lambda b: deg(b, src1, p_do1))
    pl.loop(0, EPT // 2000)(lambda b: deg(b, dst1, p_di1))
    pl.loop(0, EPT // 2000)(lambda b: deg(b, src2, p_do2))
    pl.loop(0, EPT // 2000)(lambda b: deg(b, dst2, p_di2))

    def tg(b, tok_hbm, hout):
        blk = wid + b * NW
        off = pl.multiple_of(blk * 1000, 8)
        pltpu.sync_copy(tok_hbm.at[pl.ds(off, 1000)], tid_v)
        pltpu.async_copy(embed.at[tid_v], rows_v, sem).wait()
        pltpu.sync_copy(rows_v, hout.at[pl.ds(off, 1000)])
    pl.loop(0, 100 // NW + 1)(lambda b: pl.when(wid + b * NW < 100)(
        lambda: tg(b, tok1, h1o)))
    pl.loop(0, 100 // NW + 1)(lambda b: pl.when(wid + b * NW < 100)(
        lambda: tg(b, tok2, h2o)))

    plsc.subcore_barrier()
    def wblk(b):
        blk = sid + b * NS
        off = pl.multiple_of(blk * 2000, 8)
        for p, o in ((p_do1, do1o), (p_di1, di1o), (p_do2, do2o),
                     (p_di2, di2o)):
            pltpu.sync_copy(p.at[pl.ds(off, 2000)], zb_v)
            pltpu.sync_copy(zb_v, o.at[cid, pl.ds(off, 2000)])
    pl.loop(0, 50 // NS + 1)(lambda b: pl.when(sid + b * NS < 50)(
        lambda: wblk(b)))


@functools.partial(
    pl.kernel, mesh=_MESH, compiler_params=_SC_PARAMS,
    out_type=(
        jax.ShapeDtypeStruct((N, 16), jnp.float32),
        jax.ShapeDtypeStruct((N, 16), jnp.float32),
    ),
    scratch_types=[
        pltpu.VMEM((2000, 16), jnp.float32),
        pltpu.VMEM((2000,), jnp.float32),
        pltpu.VMEM((2000,), jnp.float32),
    ],
)
def _sc_scale(h1, do1, h2, do2, hs1o, hs2o, rows_v, d_v, c_v):
    cid = lax.axis_index("c")
    sid = lax.axis_index("s")
    wid = sid * NC + cid

    def chunk(b, h_hbm, do_hbm, out_hbm):
        blk = wid + b * NW
        off = pl.multiple_of(blk * 2000, 8)
        pltpu.sync_copy(h_hbm.at[pl.ds(off, 2000)], rows_v)
        pltpu.sync_copy(do_hbm.at[0, pl.ds(off, 2000)], d_v)
        pltpu.sync_copy(do_hbm.at[1, pl.ds(off, 2000)], c_v)
        def cv(i):
            s = pl.ds(i * 16, 16)
            d = jnp.maximum(d_v[s] + c_v[s], 1.0)
            c_v[s] = _rsqrt_pos(d)
        pl.loop(0, 2000 // 16)(cv)
        def rv(i):
            cv16 = c_v[pl.ds(i * 16, 16)]
            for l in range(16):
                r = i * 16 + l
                rows_v[r, :] = rows_v[r, :] * cv16[l]
        pl.loop(0, 2000 // 16)(rv)
        pltpu.sync_copy(rows_v, out_hbm.at[pl.ds(off, 2000)])

    pl.loop(0, 50 // NW + 1)(lambda b: pl.when(wid + b * NW < 50)(
        lambda: chunk(b, h1, do1, hs1o)))
    pl.loop(0, 50 // NW + 1)(lambda b: pl.when(wid + b * NW < 50)(
        lambda: chunk(b, h2, do2, hs2o)))


NH = N // NC


@functools.partial(
    pl.kernel, mesh=_MESH, compiler_params=_SC_PARAMS,
    out_type=jax.ShapeDtypeStruct((N, 16), jnp.float32),
    scratch_types=[
        pltpu.VMEM((2000,), jnp.int32),
        pltpu.VMEM((2000,), jnp.int32),
        pltpu.VMEM((2000, 16), jnp.float32),
        pltpu.VMEM((500, 16), jnp.float32),
        pltpu.VMEM((2000,), jnp.float32),
        pltpu.VMEM((2000,), jnp.float32),
        pltpu.VMEM_SHARED((NH + 8, 16), jnp.float32),
        pltpu.SemaphoreType.DMA,
    ],
)
def _sc_gcn(hs, src, dst, di, aggo,
            sidx_v, didx_v, rows_v, zb_v, d_v, c_v, acc_s, sem):
    cid = lax.axis_index("c")
    sid = lax.axis_index("s")
    lo = cid * NH

    def z16(i):
        zb_v[i, :] = jnp.zeros((16,), jnp.float32)
    pl.loop(0, 500)(z16)

    nzb = (NH + 8) // 500 + 1
    def zblk(b):
        blk = sid + b * NS
        off = pl.multiple_of(blk * 500, 8)
        @pl.when(off < NH)
        def _():
            pltpu.sync_copy(zb_v, acc_s.at[pl.ds(off, 500)])
    pl.loop(0, nzb // NS + 1)(lambda b: pl.when(sid + b * NS < nzb)(
        lambda: zblk(b)))
    @pl.when(sid == 0)
    def _():
        pltpu.sync_copy(zb_v.at[pl.ds(0, 8)], acc_s.at[pl.ds(NH, 8)])
    plsc.subcore_barrier()

    def blkfn(b):
        off = pl.multiple_of(sid * (E // NS) + b * 2000, 8)
        pltpu.sync_copy(src.at[pl.ds(off, 2000)], sidx_v)
        cp = pltpu.async_copy(hs.at[sidx_v], rows_v, sem)
        pltpu.sync_copy(dst.at[pl.ds(off, 2000)], didx_v)
        cp.wait()
        def remap(v):
            d = didx_v[pl.ds(v * 16, 16)]
            rel = d - lo
            inr = (rel >= 0) & (rel < NH)
            didx_v[pl.ds(v * 16, 16)] = jnp.where(inr, rel, NH + (d & 7))
        pl.loop(0, 2000 // 16)(remap)
        pltpu.sync_copy(rows_v, acc_s.at[didx_v], add=True)
    pl.loop(0, E // NS // 2000)(blkfn)
    plsc.subcore_barrier()

    def wblk(b):
        blk = sid + b * NS
        off = pl.multiple_of(blk * 2000, 8)
        goff = pl.multiple_of(lo + off, 8)
        pltpu.sync_copy(di.at[0, pl.ds(goff, 2000)], d_v)
        pltpu.sync_copy(di.at[1, pl.ds(goff, 2000)], c_v)
        def cv(i):
            s = pl.ds(i * 16, 16)
            d = jnp.maximum(d_v[s] + c_v[s], 1.0)
            c_v[s] = _rsqrt_pos(d)
        pl.loop(0, 2000 // 16)(cv)
        pltpu.sync_copy(acc_s.at[pl.ds(off, 2000)], rows_v)
        def rv(i):
            cv16 = c_v[pl.ds(i * 16, 16)]
            for l in range(16):
                r = i * 16 + l
                rows_v[r, :] = rows_v[r, :] * cv16[l]
        pl.loop(0, 2000 // 16)(rv)
        pltpu.sync_copy(rows_v, aggo.at[pl.ds(goff, 2000)])
    nwb = NH // 2000
    pl.loop(0, nwb // NS + 1)(lambda b: pl.when(sid + b * NS < nwb)(
        lambda: wblk(b)))


@functools.partial(
    pl.kernel, mesh=_MESH, compiler_params=_SC_PARAMS,
    out_type=(
        jax.ShapeDtypeStruct((N * 32,), jnp.float32),
        jax.ShapeDtypeStruct((NW, SPILL_CAP), jnp.int32),
        jax.ShapeDtypeStruct((NW, SPILL_CAP), jnp.int32),
    ),
    scratch_types=[
        pltpu.VMEM((NPT * 32 + 32,), jnp.float32),
        pltpu.VMEM((2000,), jnp.int32),
        pltpu.VMEM((2000,), jnp.int32),
        pltpu.VMEM((2000,), jnp.int32),
        pltpu.VMEM((2000,), jnp.int32),
        pltpu.VMEM((2016,), jnp.int32),
        pltpu.VMEM((2016,), jnp.int32),
        pltpu.VMEM((128,), jnp.int32),
        pltpu.VMEM((128,), jnp.int32),
        pltpu.VMEM((128,), jnp.int32),
        pltpu.VMEM((128,), jnp.int32),
        pltpu.VMEM((128, 32), jnp.float32),
        pltpu.VMEM((128, 32), jnp.float32),
        pltpu.VMEM((3136,), jnp.int32),
        pltpu.VMEM((128,), jnp.int32),
        pltpu.SemaphoreType.DMA,
        pltpu.SemaphoreType.DMA,
        pltpu.SemaphoreType.DMA,
        pltpu.SemaphoreType.DMA,
        pltpu.SemaphoreType.DMA,
        pltpu.SemaphoreType.DMA,
    ],
)
def _sc_max(hp, src, dst, mxo, ssrco, sdloco,
            acc_v, db0, db1, sb0, sb1, sels_v, seld_v,
            bs0, bs1, bd0, bd1, rows0, rows1,
            tmp_v, cntb_v, semd0, semd1, sems0, sems1, semg0, semg1):
    cid = lax.axis_index("c")
    sid = lax.axis_index("s")
    wid = sid * NC + cid
    lo = wid * NPT
    iota = lax.iota(jnp.int32, 16)
    db = (db0, db1)
    sb = (sb0, sb1)
    semd = (semd0, semd1)
    sems = (sems0, sems1)
    bs = (bs0, bs1)
    bd = (bd0, bd1)
    rows = (rows0, rows1)
    semg = (semg0, semg1)

    def zacc(i):
        acc_v[pl.ds(i * 16, 16)] = jnp.zeros((16,), jnp.float32)
    pl.loop(0, (NPT * 32 + 32) // 16)(zacc)

    trash = jnp.full((16,), NPT, jnp.int32)
    NBLK = E // 2000
    unpt = jnp.uint32(NPT)

    def issue(blk, par):
        off = pl.multiple_of(blk * 2000, 8)
        pltpu.async_copy(dst.at[pl.ds(off, 2000)], db[par], semd[par])
        pltpu.async_copy(src.at[pl.ds(off, 2000)], sb[par], sems[par])

    issue(0, 0)
    issue(1, 1)

    lane0 = iota == 0

    def scan_half(blk, par, gcur):
        off = pl.multiple_of(blk * 2000, 8)
        pltpu.make_async_copy(dst.at[pl.ds(off, 2000)], db[par],
                              semd[par]).wait()
        pltpu.make_async_copy(src.at[pl.ds(off, 2000)], sb[par],
                              sems[par]).wait()
        cntb_v[pl.ds(112, 16)] = jnp.zeros((16,), jnp.int32)
        for v in range(125):
            d = db[par][pl.ds(v * 16, 16)]
            m = plsc.bitcast(d - lo, jnp.uint32) < unpt
            pc = plsc.all_reduce_population_count(m)
            plsc.store_scatter(cntb_v, [jnp.full((16,), v, jnp.int32)],
                               pc, mask=lane0)
        carry = jnp.int32(0)
        prefs = []
        for g in range(8):
            grp = cntb_v[pl.ds(g * 16, 16)]
            cs = plsc.cumsum(grp)
            prefs.append(cs - grp + carry)
            carry = carry + cs[15]
        for v in range(125):
            offv = prefs[v // 16][v % 16]
            d = db[par][pl.ds(v * 16, 16)]
            rel = d - lo
            m = plsc.bitcast(rel, jnp.uint32) < unpt
            sv = sb[par][pl.ds(v * 16, 16)]
            plsc.store_compressed(sels_v.at[pl.ds(offv, 16)], sv, mask=m)
            plsc.store_compressed(seld_v.at[pl.ds(offv, 16)], rel, mask=m)
        cnt = carry

        @pl.when(blk + 2 < NBLK)
        def _():
            issue(blk + 2, par)

        sels_v[pl.ds(cnt, 16)] = iota
        seld_v[pl.ds(cnt, 16)] = trash
        cnt16 = (cnt + 15) & ~15
        nfl = (cnt16 + 255) // 256
        def flush(f, _):
            foff = pl.multiple_of(f * 256, 8)
            goff = pl.multiple_of(gcur + foff, 8)
            pltpu.sync_copy(sels_v.at[pl.ds(foff, 256)],
                            ssrco.at[wid, pl.ds(goff, 256)])
            pltpu.sync_copy(seld_v.at[pl.ds(foff, 256)],
                            sdloco.at[wid, pl.ds(goff, 256)])
            return 0
        lax.fori_loop(0, nfl, flush, 0, unroll=False)
        return gcur + cnt16

    def scan_pair(b2, gcur):
        gcur = scan_half(2 * b2, 0, gcur)
        gcur = scan_half(2 * b2 + 1, 1, gcur)
        return gcur

    gcur = lax.fori_loop(0, NBLK // 2, scan_pair, jnp.int32(0), unroll=False)

    def padfn(i):
        sels_v[pl.ds(i * 16, 16)] = iota
        seld_v[pl.ds(i * 16, 16)] = trash
    pl.loop(0, 16)(padfn)
    gpad = pl.multiple_of(gcur, 8)
    pltpu.sync_copy(sels_v.at[pl.ds(0, 256)], ssrco.at[wid, pl.ds(gpad, 256)])
    pltpu.sync_copy(seld_v.at[pl.ds(0, 256)], sdloco.at[wid, pl.ds(gpad, 256)])

    nb = (gcur + 127) // 128

    def fetch(j, par):
        boff = pl.multiple_of(j * 128, 8)
        pltpu.sync_copy(ssrco.at[wid, pl.ds(boff, 128)], bs[par])
        pltpu.sync_copy(sdloco.at[wid, pl.ds(boff, 128)], bd[par])
        pltpu.async_copy(hp.at[bs[par]], rows[par], semg[par])

    @pl.when(nb > 0)
    def _():
        fetch(0, 0)

    def batch_half(j, par):
        @pl.when(j + 1 < nb)
        def _():
            fetch(j + 1, 1 - par)
        pltpu.make_async_copy(hp.at[bs[par]], rows[par], semg[par]).wait()

        def vreg(v, _):
            dl = bd[par][pl.ds(v * 16, 16)]
            a0 = dl * 32
            rs = v * 16 + iota
            plsc.store_scatter(tmp_v, [dl], iota)
            rd = plsc.load_gather(tmp_v, [dl])
            ndup = jnp.sum((rd != iota).astype(jnp.int32))

            @pl.when(ndup == 0)
            def _():
                for f in range(32):
                    fv = jnp.full((16,), f, jnp.int32)
                    cur = plsc.load_gather(acc_v, [a0 + f])
                    val = plsc.load_gather(rows[par], [rs, fv])
                    plsc.store_scatter(acc_v, [a0 + f], jnp.maximum(cur, val))

            @pl.when(ndup != 0)
            def _():
                def cond(carry):
                    pend, _ = carry
                    return jnp.sum(pend.astype(jnp.int32)) > 0

                def body(carry):
                    pend, it = carry
                    fail = jnp.zeros((16,), jnp.bool_)
                    for f in range(32):
                        fv = jnp.full((16,), f, jnp.int32)
                        cur = plsc.load_gather(acc_v, [a0 + f], mask=pend)
                        val = plsc.load_gather(rows[par], [rs, fv], mask=pend)
                        nw = jnp.maximum(cur, val)
                        plsc.store_scatter(acc_v, [a0 + f], nw, mask=pend)
                        chk = plsc.load_gather(acc_v, [a0 + f], mask=pend)
                        fail = fail | ((chk < nw) & pend)
                    return fail, it + 1

                pend0 = jnp.ones((16,), jnp.bool_)
                lax.while_loop(cond, body, (pend0, jnp.int32(0)))
            return 0

        lax.fori_loop(0, 8, vreg, 0, unroll=False)

    def batch_pair(j2, _):
        j = 2 * j2
        @pl.when(j < nb)
        def _():
            batch_half(j, 0)
        @pl.when(j + 1 < nb)
        def _():
            batch_half(j + 1, 1)
        return 0

    lax.fori_loop(0, (nb + 1) // 2, batch_pair, 0, unroll=False)

    moff = pl.multiple_of(wid * (NPT * 32), 8)
    pltpu.sync_copy(acc_v.at[pl.ds(0, NPT * 32)], mxo.at[pl.ds(moff, NPT * 32)])


def _tc_h1_body(agg_ref, wg_ref, bg_ref, wp_ref, bp_ref, h1_ref, hp_ref):
    agg = agg_ref[...]
    h1 = jnp.maximum(
        jnp.dot(agg, wg_ref[...], preferred_element_type=jnp.float32)
        + bg_ref[...], 0.0)
    hp = jnp.maximum(
        jnp.dot(h1, wp_ref[...], preferred_element_type=jnp.float32)
        + bp_ref[...], 0.0)
    h1_ref[...] = h1
    hp_ref[...] = hp


def _tc_h1(agg, wg, bg, wp, bp):
    B = 2000
    grid = N // B
    return pl.pallas_call(
        _tc_h1_body,
        grid=(grid,),
        in_specs=[
            pl.BlockSpec((B, 16), lambda i: (i, 0)),
            pl.BlockSpec((16, 32), lambda i: (0, 0)),
            pl.BlockSpec((1, 32), lambda i: (0, 0)),
            pl.BlockSpec((32, 32), lambda i: (0, 0)),
            pl.BlockSpec((1, 32), lambda i: (0, 0)),
        ],
        out_specs=[
            pl.BlockSpec((B, 32), lambda i: (i, 0)),
            pl.BlockSpec((B, 32), lambda i: (i, 0)),
        ],
        out_shape=[
            jax.ShapeDtypeStruct((N, 32), jnp.float32),
            jax.ShapeDtypeStruct((N, 32), jnp.float32),
        ],
    )(agg, wg, bg, wp, bp)


def _tc_tail_body(h1_ref, mx_ref, gid_ref, ws_ref, wn_ref, bs_ref,
                  we_ref, be_ref, hg_ref):
    i = pl.program_id(0)
    h2 = jnp.maximum(
        jnp.dot(h1_ref[...], ws_ref[...], preferred_element_type=jnp.float32)
        + jnp.dot(mx_ref[...], wn_ref[...], preferred_element_type=jnp.float32)
        + bs_ref[...], 0.0)
    h3 = jnp.maximum(
        jnp.dot(h2, we_ref[...], preferred_element_type=jnp.float32)
        + be_ref[...], 0.0)
    B = h3.shape[0]
    gid = gid_ref[...]
    onehot = (gid == lax.broadcasted_iota(jnp.int32, (G, B), 0)).astype(
        jnp.float32)
    hg = lax.dot_general(onehot, h3, (((1,), (0,)), ((), ())),
                         preferred_element_type=jnp.float32)

    @pl.when(i == 0)
    def _():
        hg_ref[...] = jnp.zeros_like(hg_ref)
    hg_ref[...] += hg


def _tc_tail(h1, mx, gid2d, ws, wn, bs, we, be):
    B = 2048
    NP = 102400
    h1 = jnp.pad(h1, ((0, NP - N), (0, 0)))
    mx = jnp.pad(mx, ((0, NP - N), (0, 0)))
    gid2d = jnp.pad(gid2d, ((0, 0), (0, NP - N)), constant_values=G + 7)
    grid = NP // B
    return pl.pallas_call(
        _tc_tail_body,
        grid=(grid,),
        in_specs=[
            pl.BlockSpec((B, 32), lambda i: (i, 0)),
            pl.BlockSpec((B, 32), lambda i: (i, 0)),
            pl.BlockSpec((1, B), lambda i: (0, i)),
            pl.BlockSpec((32, 64), lambda i: (0, 0)),
            pl.BlockSpec((32, 64), lambda i: (0, 0)),
            pl.BlockSpec((1, 64), lambda i: (0, 0)),
            pl.BlockSpec((64, 64), lambda i: (0, 0)),
            pl.BlockSpec((1, 64), lambda i: (0, 0)),
        ],
        out_specs=pl.BlockSpec((G, 64), lambda i: (0, 0)),
        out_shape=jax.ShapeDtypeStruct((G, 64), jnp.float32),
    )(h1, mx, gid2d, ws, wn, bs, we, be)


def _top_kernel(hg1_ref, hg2_ref, wt_ref, bt_ref, out_ref):
    z1 = jnp.maximum(
        jnp.dot(hg1_ref[...], wt_ref[...], preferred_element_type=jnp.float32)
        + bt_ref[...], 0.0)
    z2 = jnp.maximum(
        jnp.dot(hg2_ref[...], wt_ref[...], preferred_element_type=jnp.float32)
        + bt_ref[...], 0.0)
    num = jnp.sum(z1 * z2, axis=1)
    n1 = jnp.sqrt(jnp.sum(z1 * z1, axis=1))
    n2 = jnp.sqrt(jnp.sum(z2 * z2, axis=1))
    sim = num / jnp.maximum(n1 * n2, 1e-8)
    out_ref[...] = jnp.clip(jnp.abs(sim), 0.0, 1.0)


def kernel(tokens1, edge_index1, graph_ids1, tokens2, edge_index2, graph_ids2,
           embed, W_gcn, b_gcn, W_pool, b_pool, W_self, W_neigh, b_sage,
           W_enc, b_enc, W_top, b_top):
    i32 = jnp.int32
    tok1 = tokens1.astype(i32)
    tok2 = tokens2.astype(i32)
    src1 = edge_index1[0].astype(i32)
    dst1 = edge_index1[1].astype(i32)
    src2 = edge_index2[0].astype(i32)
    dst2 = edge_index2[1].astype(i32)
    gid1 = graph_ids1.astype(i32).reshape(1, N)
    gid2 = graph_ids2.astype(i32).reshape(1, N)

    h1, h2, do1, di1, do2, di2 = _sc_prep(
        tok1, tok2, embed, src1, dst1, src2, dst2)
    hs1, hs2 = _sc_scale(h1, do1, h2, do2)
    agg1 = _sc_gcn(hs1, src1, dst1, di1)
    agg2 = _sc_gcn(hs2, src2, dst2, di2)

    bg = b_gcn.reshape(1, 32)
    bp = b_pool.reshape(1, 32)
    h1a, hp1 = _tc_h1(agg1, W_gcn, bg, W_pool, bp)
    h1b, hp2 = _tc_h1(agg2, W_gcn, bg, W_pool, bp)

    mx1f, _, _ = _sc_max(hp1, src1, dst1)
    mx2f, _, _ = _sc_max(hp2, src2, dst2)
    mx1 = mx1f.reshape(N, 32)
    mx2 = mx2f.reshape(N, 32)

    bs = b_sage.reshape(1, 64)
    be = b_enc.reshape(1, 64)
    hg1 = _tc_tail(h1a, mx1, gid1, W_self, W_neigh, bs, W_enc, be)
    hg2 = _tc_tail(h1b, mx2, gid2, W_self, W_neigh, bs, W_enc, be)

    sim = pl.pallas_call(
        _top_kernel,
        out_shape=jax.ShapeDtypeStruct((G,), jnp.float32),
    )(hg1, hg2, W_top, b_top.reshape(1, 128))
    return sim


# speedup vs baseline: 1.1664x; 1.1664x over previous
import functools

import jax
import jax.numpy as jnp
from jax import lax
from jax.experimental import pallas as pl
from jax.experimental.pallas import tpu as pltpu, tpu_sc as plsc

N = 100000
E = 1600000
G = 128
VOCAB = 8018

NC = 2
NS = 16
NW = NC * NS
NPT = N // NW
EPT = E // NW

_MESH = plsc.VectorSubcoreMesh(core_axis_name="c", subcore_axis_name="s")
_SC_PARAMS = pltpu.CompilerParams(
    needs_layout_passes=False, use_tc_tiling_on_sc=False)

SPILL_CAP = E + 4096


def _rsqrt_pos(x):
    i = plsc.bitcast(x, jnp.int32)
    y = plsc.bitcast(jnp.int32(0x5F3759DF) - (i >> 1), jnp.float32)
    for _ in range(3):
        y = y * (1.5 - 0.5 * x * y * y)
    return y


@functools.partial(
    pl.kernel, mesh=_MESH, compiler_params=_SC_PARAMS,
    out_type=(
        jax.ShapeDtypeStruct((N, 16), jnp.float32),
        jax.ShapeDtypeStruct((N, 16), jnp.float32),
        jax.ShapeDtypeStruct((2, N), jnp.float32),
        jax.ShapeDtypeStruct((2, N), jnp.float32),
        jax.ShapeDtypeStruct((2, N), jnp.float32),
        jax.ShapeDtypeStruct((2, N), jnp.float32),
    ),
    scratch_types=[
        pltpu.VMEM((1000,), jnp.int32),
        pltpu.VMEM((1000, 16), jnp.float32),
        pltpu.VMEM((2000,), jnp.int32),
        pltpu.VMEM((2000,), jnp.float32),
        pltpu.VMEM((2000,), jnp.float32),
        pltpu.VMEM_SHARED((N,), jnp.float32),
        pltpu.VMEM_SHARED((N,), jnp.float32),
        pltpu.VMEM_SHARED((N,), jnp.float32),
        pltpu.VMEM_SHARED((N,), jnp.float32),
        pltpu.SemaphoreType.DMA,
    ],
)
def _sc_prep(tok1, tok2, embed, src1, dst1, src2, dst2,
             h1o, h2o, do1o, di1o, do2o, di2o,
             tid_v, rows_v, eid_v, ones_v, zb_v,
             p_do1, p_di1, p_do2, p_di2, sem):
    cid = lax.axis_index("c")
    sid = lax.axis_index("s")
    wid = sid * NC + cid

    def z16(i):
        zb_v[pl.ds(i * 16, 16)] = jnp.zeros((16,), jnp.float32)
        ones_v[pl.ds(i * 16, 16)] = jnp.ones((16,), jnp.float32)
    pl.loop(0, 2000 // 16)(z16)

    def zblk(b):
        blk = sid + b * NS
        off = pl.multiple_of(blk * 2000, 8)
        for p in (p_do1, p_di1, p_do2, p_di2):
            pltpu.sync_copy(zb_v, p.at[pl.ds(off, 2000)])
    pl.loop(0, 50 // NS + 1)(lambda b: pl.when(sid + b * NS < 50)(
        lambda: zblk(b)))
    plsc.subcore_barrier()

    def deg(b, idx_hbm, plane):
        off = pl.multiple_of(wid * EPT + b * 2000, 8)
        pltpu.sync_copy(idx_hbm.at[pl.ds(off, 2000)], eid_v)
        pltpu.sync_copy(ones_v, plane.at[eid_v], add=True)
    pl.loop(0, EPT // 2000)(lambda b: deg(b, src1, p_do1))
    pl.loop(0, EPT // 2000)(lambda b: deg(b, dst1, p_di1))
    pl.loop(0, EPT // 2000)(lambda b: deg(b, src2, p_do2))
    pl.loop(0, EPT // 2000)(lambda b: deg(b, dst2, p_di2))

    def tg(b, tok_hbm, hout):
        blk = wid + b * NW
        off = pl.multiple_of(blk * 1000, 8)
        pltpu.sync_copy(tok_hbm.at[pl.ds(off, 1000)], tid_v)
        pltpu.async_copy(embed.at[tid_v], rows_v, sem).wait()
        pltpu.sync_copy(rows_v, hout.at[pl.ds(off, 1000)])
    pl.loop(0, 100 // NW + 1)(lambda b: pl.when(wid + b * NW < 100)(
        lambda: tg(b, tok1, h1o)))
    pl.loop(0, 100 // NW + 1)(lambda b: pl.when(wid + b * NW < 100)(
        lambda: tg(b, tok2, h2o)))

    plsc.subcore_barrier()
    def wblk(b):
        blk = sid + b * NS
        off = pl.multiple_of(blk * 2000, 8)
        for p, o in ((p_do1, do1o), (p_di1, di1o), (p_do2, do2o),
                     (p_di2, di2o)):
            pltpu.sync_copy(p.at[pl.ds(off, 2000)], zb_v)
            pltpu.sync_copy(zb_v, o.at[cid, pl.ds(off, 2000)])
    pl.loop(0, 50 // NS + 1)(lambda b: pl.when(sid + b * NS < 50)(
        lambda: wblk(b)))


@functools.partial(
    pl.kernel, mesh=_MESH, compiler_params=_SC_PARAMS,
    out_type=(
        jax.ShapeDtypeStruct((N, 16), jnp.float32),
        jax.ShapeDtypeStruct((N, 16), jnp.float32),
    ),
    scratch_types=[
        pltpu.VMEM((2000, 16), jnp.float32),
        pltpu.VMEM((2000,), jnp.float32),
        pltpu.VMEM((2000,), jnp.float32),
    ],
)
def _sc_scale(h1, do1, h2, do2, hs1o, hs2o, rows_v, d_v, c_v):
    cid = lax.axis_index("c")
    sid = lax.axis_index("s")
    wid = sid * NC + cid

    def chunk(b, h_hbm, do_hbm, out_hbm):
        blk = wid + b * NW
        off = pl.multiple_of(blk * 2000, 8)
        pltpu.sync_copy(h_hbm.at[pl.ds(off, 2000)], rows_v)
        pltpu.sync_copy(do_hbm.at[0, pl.ds(off, 2000)], d_v)
        pltpu.sync_copy(do_hbm.at[1, pl.ds(off, 2000)], c_v)
        def cv(i):
            s = pl.ds(i * 16, 16)
            d = jnp.maximum(d_v[s] + c_v[s], 1.0)
            c_v[s] = _rsqrt_pos(d)
        pl.loop(0, 2000 // 16)(cv)
        def rv(i):
            cv16 = c_v[pl.ds(i * 16, 16)]
            for l in range(16):
                r = i * 16 + l
                rows_v[r, :] = rows_v[r, :] * cv16[l]
        pl.loop(0, 2000 // 16)(rv)
        pltpu.sync_copy(rows_v, out_hbm.at[pl.ds(off, 2000)])

    pl.loop(0, 50 // NW + 1)(lambda b: pl.when(wid + b * NW < 50)(
        lambda: chunk(b, h1, do1, hs1o)))
    pl.loop(0, 50 // NW + 1)(lambda b: pl.when(wid + b * NW < 50)(
        lambda: chunk(b, h2, do2, hs2o)))


NH = N // NC


@functools.partial(
    pl.kernel, mesh=_MESH, compiler_params=_SC_PARAMS,
    out_type=jax.ShapeDtypeStruct((N, 16), jnp.float32),
    scratch_types=[
        pltpu.VMEM((2000,), jnp.int32),
        pltpu.VMEM((2000,), jnp.int32),
        pltpu.VMEM((2000, 16), jnp.float32),
        pltpu.VMEM((500, 16), jnp.float32),
        pltpu.VMEM((2000,), jnp.float32),
        pltpu.VMEM((2000,), jnp.float32),
        pltpu.VMEM_SHARED((NH + 8, 16), jnp.float32),
        pltpu.SemaphoreType.DMA,
    ],
)
def _sc_gcn(hs, src, dst, di, aggo,
            sidx_v, didx_v, rows_v, zb_v, d_v, c_v, acc_s, sem):
    cid = lax.axis_index("c")
    sid = lax.axis_index("s")
    lo = cid * NH

    def z16(i):
        zb_v[i, :] = jnp.zeros((16,), jnp.float32)
    pl.loop(0, 500)(z16)

    nzb = (NH + 8) // 500 + 1
    def zblk(b):
        blk = sid + b * NS
        off = pl.multiple_of(blk * 500, 8)
        @pl.when(off < NH)
        def _():
            pltpu.sync_copy(zb_v, acc_s.at[pl.ds(off, 500)])
    pl.loop(0, nzb // NS + 1)(lambda b: pl.when(sid + b * NS < nzb)(
        lambda: zblk(b)))
    @pl.when(sid == 0)
    def _():
        pltpu.sync_copy(zb_v.at[pl.ds(0, 8)], acc_s.at[pl.ds(NH, 8)])
    plsc.subcore_barrier()

    def blkfn(b):
        off = pl.multiple_of(sid * (E // NS) + b * 2000, 8)
        pltpu.sync_copy(src.at[pl.ds(off, 2000)], sidx_v)
        cp = pltpu.async_copy(hs.at[sidx_v], rows_v, sem)
        pltpu.sync_copy(dst.at[pl.ds(off, 2000)], didx_v)
        cp.wait()
        def remap(v):
            d = didx_v[pl.ds(v * 16, 16)]
            rel = d - lo
            inr = (rel >= 0) & (rel < NH)
            didx_v[pl.ds(v * 16, 16)] = jnp.where(inr, rel, NH + (d & 7))
        pl.loop(0, 2000 // 16)(remap)
        pltpu.sync_copy(rows_v, acc_s.at[didx_v], add=True)
    pl.loop(0, E // NS // 2000)(blkfn)
    plsc.subcore_barrier()

    def wblk(b):
        blk = sid + b * NS
        off = pl.multiple_of(blk * 2000, 8)
        goff = pl.multiple_of(lo + off, 8)
        pltpu.sync_copy(di.at[0, pl.ds(goff, 2000)], d_v)
        pltpu.sync_copy(di.at[1, pl.ds(goff, 2000)], c_v)
        def cv(i):
            s = pl.ds(i * 16, 16)
            d = jnp.maximum(d_v[s] + c_v[s], 1.0)
            c_v[s] = _rsqrt_pos(d)
        pl.loop(0, 2000 // 16)(cv)
        pltpu.sync_copy(acc_s.at[pl.ds(off, 2000)], rows_v)
        def rv(i):
            cv16 = c_v[pl.ds(i * 16, 16)]
            for l in range(16):
                r = i * 16 + l
                rows_v[r, :] = rows_v[r, :] * cv16[l]
        pl.loop(0, 2000 // 16)(rv)
        pltpu.sync_copy(rows_v, aggo.at[pl.ds(goff, 2000)])
    nwb = NH // 2000
    pl.loop(0, nwb // NS + 1)(lambda b: pl.when(sid + b * NS < nwb)(
        lambda: wblk(b)))


@functools.partial(
    pl.kernel, mesh=_MESH, compiler_params=_SC_PARAMS,
    out_type=(
        jax.ShapeDtypeStruct((N * 32,), jnp.float32),
        jax.ShapeDtypeStruct((NW, SPILL_CAP), jnp.int32),
        jax.ShapeDtypeStruct((NW, SPILL_CAP), jnp.int32),
    ),
    scratch_types=[
        pltpu.VMEM((NPT * 32 + 32,), jnp.float32),
        pltpu.VMEM((2000,), jnp.int32),
        pltpu.VMEM((2000,), jnp.int32),
        pltpu.VMEM((2000,), jnp.int32),
        pltpu.VMEM((2000,), jnp.int32),
        pltpu.VMEM((2016,), jnp.int32),
        pltpu.VMEM((2016,), jnp.int32),
        pltpu.VMEM((128,), jnp.int32),
        pltpu.VMEM((128,), jnp.int32),
        pltpu.VMEM((128,), jnp.int32),
        pltpu.VMEM((128,), jnp.int32),
        pltpu.VMEM((128, 32), jnp.float32),
        pltpu.VMEM((128, 32), jnp.float32),
        pltpu.VMEM((3136,), jnp.int32),
        pltpu.VMEM((128,), jnp.int32),
        pltpu.SemaphoreType.DMA,
        pltpu.SemaphoreType.DMA,
        pltpu.SemaphoreType.DMA,
        pltpu.SemaphoreType.DMA,
        pltpu.SemaphoreType.DMA,
        pltpu.SemaphoreType.DMA,
    ],
)
def _sc_max(hp, src, dst, mxo, ssrco, sdloco,
            acc_v, db0, db1, sb0, sb1, sels_v, seld_v,
            bs0, bs1, bd0, bd1, rows0, rows1,
            tmp_v, cntb_v, semd0, semd1, sems0, sems1, semg0, semg1):
    cid = lax.axis_index("c")
    sid = lax.axis_index("s")
    wid = sid * NC + cid
    lo = wid * NPT
    iota = lax.iota(jnp.int32, 16)
    db = (db0, db1)
    sb = (sb0, sb1)
    semd = (semd0, semd1)
    sems = (sems0, sems1)
    bs = (bs0, bs1)
    bd = (bd0, bd1)
    rows = (rows0, rows1)
    semg = (semg0, semg1)

    def zacc(i):
        acc_v[pl.ds(i * 16, 16)] = jnp.zeros((16,), jnp.float32)
    pl.loop(0, (NPT * 32 + 32) // 16)(zacc)

    trash = jnp.full((16,), NPT, jnp.int32)
    NBLK = E // 2000
    unpt = jnp.uint32(NPT)

    def issue(blk, par):
        off = pl.multiple_of(blk * 2000, 8)
        pltpu.async_copy(dst.at[pl.ds(off, 2000)], db[par], semd[par])
        pltpu.async_copy(src.at[pl.ds(off, 2000)], sb[par], sems[par])

    issue(0, 0)
    issue(1, 1)

    def scan_half(blk, par, gcur):
        off = pl.multiple_of(blk * 2000, 8)
        pltpu.make_async_copy(dst.at[pl.ds(off, 2000)], db[par],
                              semd[par]).wait()
        pltpu.make_async_copy(src.at[pl.ds(off, 2000)], sb[par],
                              sems[par]).wait()
        cnt = jnp.int32(0)
        for v in range(125):
            d = db[par][pl.ds(v * 16, 16)]
            rel = d - lo
            m = plsc.bitcast(rel, jnp.uint32) < unpt
            sv = sb[par][pl.ds(v * 16, 16)]
            plsc.store_compressed(sels_v.at[pl.ds(cnt, 16)], sv, mask=m)
            plsc.store_compressed(seld_v.at[pl.ds(cnt, 16)], rel, mask=m)
            pc = plsc.all_reduce_population_count(m)
            cnt = cnt + pc[0]

        @pl.when(blk + 2 < NBLK)
        def _():
            issue(blk + 2, par)

        sels_v[pl.ds(cnt, 16)] = iota
        seld_v[pl.ds(cnt, 16)] = trash
        cnt16 = (cnt + 15) & ~15
        nfl = (cnt16 + 255) // 256
        def flush(f, _):
            foff = pl.multiple_of(f * 256, 8)
            goff = pl.multiple_of(gcur + foff, 8)
            pltpu.sync_copy(sels_v.at[pl.ds(foff, 256)],
                            ssrco.at[wid, pl.ds(goff, 256)])
            pltpu.sync_copy(seld_v.at[pl.ds(foff, 256)],
                            sdloco.at[wid, pl.ds(goff, 256)])
            return 0
        lax.fori_loop(0, nfl, flush, 0, unroll=False)
        return gcur + cnt16

    def scan_pair(b2, gcur):
        gcur = scan_half(2 * b2, 0, gcur)
        gcur = scan_half(2 * b2 + 1, 1, gcur)
        return gcur

    gcur = lax.fori_loop(0, NBLK // 2, scan_pair, jnp.int32(0), unroll=False)

    def padfn(i):
        sels_v[pl.ds(i * 16, 16)] = iota
        seld_v[pl.ds(i * 16, 16)] = trash
    pl.loop(0, 16)(padfn)
    gpad = pl.multiple_of(gcur, 8)
    pltpu.sync_copy(sels_v.at[pl.ds(0, 256)], ssrco.at[wid, pl.ds(gpad, 256)])
    pltpu.sync_copy(seld_v.at[pl.ds(0, 256)], sdloco.at[wid, pl.ds(gpad, 256)])

    nb = (gcur + 127) // 128

    def fetch_idx(j, par):
        boff = pl.multiple_of(j * 128, 8)
        pltpu.async_copy(ssrco.at[wid, pl.ds(boff, 128)], bs[par], semd[par])
        pltpu.async_copy(sdloco.at[wid, pl.ds(boff, 128)], bd[par], sems[par])

    def wait_idx(j, par):
        boff = pl.multiple_of(j * 128, 8)
        pltpu.make_async_copy(ssrco.at[wid, pl.ds(boff, 128)], bs[par],
                              semd[par]).wait()
        pltpu.make_async_copy(sdloco.at[wid, pl.ds(boff, 128)], bd[par],
                              sems[par]).wait()

    @pl.when(nb > 0)
    def _():
        fetch_idx(0, 0)
        wait_idx(0, 0)
        pltpu.async_copy(hp.at[bs[0]], rows[0], semg[0])
        @pl.when(nb > 1)
        def _():
            fetch_idx(1, 1)

    def batch_half(j, par):
        @pl.when(j + 1 < nb)
        def _():
            wait_idx(j + 1, 1 - par)
            pltpu.async_copy(hp.at[bs[1 - par]], rows[1 - par],
                             semg[1 - par])
        pltpu.make_async_copy(hp.at[bs[par]], rows[par], semg[par]).wait()

        def vreg(v, _):
            dl = bd[par][pl.ds(v * 16, 16)]
            a0 = dl * 32
            rs = v * 16 + iota
            plsc.store_scatter(tmp_v, [dl], iota)
            rd = plsc.load_gather(tmp_v, [dl])
            ndup = jnp.sum((rd != iota).astype(jnp.int32))

            @pl.when(ndup == 0)
            def _():
                for f in range(32):
                    fv = jnp.full((16,), f, jnp.int32)
                    cur = plsc.load_gather(acc_v, [a0 + f])
                    val = plsc.load_gather(rows[par], [rs, fv])
                    plsc.store_scatter(acc_v, [a0 + f], jnp.maximum(cur, val))

            @pl.when(ndup != 0)
            def _():
                def cond(carry):
                    pend, _ = carry
                    return jnp.sum(pend.astype(jnp.int32)) > 0

                def body(carry):
                    pend, it = carry
                    fail = jnp.zeros((16,), jnp.bool_)
                    for f in range(32):
                        fv = jnp.full((16,), f, jnp.int32)
                        cur = plsc.load_gather(acc_v, [a0 + f], mask=pend)
                        val = plsc.load_gather(rows[par], [rs, fv], mask=pend)
                        nw = jnp.maximum(cur, val)
                        plsc.store_scatter(acc_v, [a0 + f], nw, mask=pend)
                        chk = plsc.load_gather(acc_v, [a0 + f], mask=pend)
                        fail = fail | ((chk < nw) & pend)
                    return fail, it + 1

                pend0 = jnp.ones((16,), jnp.bool_)
                lax.while_loop(cond, body, (pend0, jnp.int32(0)))
            return 0

        lax.fori_loop(0, 8, vreg, 0, unroll=False)
        @pl.when(j + 2 < nb)
        def _():
            fetch_idx(j + 2, par)

    def batch_pair(j2, _):
        j = 2 * j2
        @pl.when(j < nb)
        def _():
            batch_half(j, 0)
        @pl.when(j + 1 < nb)
        def _():
            batch_half(j + 1, 1)
        return 0

    lax.fori_loop(0, (nb + 1) // 2, batch_pair, 0, unroll=False)

    moff = pl.multiple_of(wid * (NPT * 32), 8)
    pltpu.sync_copy(acc_v.at[pl.ds(0, NPT * 32)], mxo.at[pl.ds(moff, NPT * 32)])


def _tc_h1_body(agg_ref, wg_ref, bg_ref, wp_ref, bp_ref, h1_ref, hp_ref):
    agg = agg_ref[...]
    h1 = jnp.maximum(
        jnp.dot(agg, wg_ref[...], preferred_element_type=jnp.float32)
        + bg_ref[...], 0.0)
    hp = jnp.maximum(
        jnp.dot(h1, wp_ref[...], preferred_element_type=jnp.float32)
        + bp_ref[...], 0.0)
    h1_ref[...] = h1
    hp_ref[...] = hp


def _tc_h1(agg, wg, bg, wp, bp):
    B = 2000
    grid = N // B
    return pl.pallas_call(
        _tc_h1_body,
        grid=(grid,),
        in_specs=[
            pl.BlockSpec((B, 16), lambda i: (i, 0)),
            pl.BlockSpec((16, 32), lambda i: (0, 0)),
            pl.BlockSpec((1, 32), lambda i: (0, 0)),
            pl.BlockSpec((32, 32), lambda i: (0, 0)),
            pl.BlockSpec((1, 32), lambda i: (0, 0)),
        ],
        out_specs=[
            pl.BlockSpec((B, 32), lambda i: (i, 0)),
            pl.BlockSpec((B, 32), lambda i: (i, 0)),
        ],
        out_shape=[
            jax.ShapeDtypeStruct((N, 32), jnp.float32),
            jax.ShapeDtypeStruct((N, 32), jnp.float32),
        ],
    )(agg, wg, bg, wp, bp)


def _tc_tail_body(h1_ref, mx_ref, gid_ref, ws_ref, wn_ref, bs_ref,
                  we_ref, be_ref, hg_ref):
    i = pl.program_id(0)
    h2 = jnp.maximum(
        jnp.dot(h1_ref[...], ws_ref[...], preferred_element_type=jnp.float32)
        + jnp.dot(mx_ref[...], wn_ref[...], preferred_element_type=jnp.float32)
        + bs_ref[...], 0.0)
    h3 = jnp.maximum(
        jnp.dot(h2, we_ref[...], preferred_element_type=jnp.float32)
        + be_ref[...], 0.0)
    B = h3.shape[0]
    gid = gid_ref[...]
    onehot = (gid == lax.broadcasted_iota(jnp.int32, (G, B), 0)).astype(
        jnp.float32)
    hg = lax.dot_general(onehot, h3, (((1,), (0,)), ((), ())),
                         preferred_element_type=jnp.float32)

    @pl.when(i == 0)
    def _():
        hg_ref[...] = jnp.zeros_like(hg_ref)
    hg_ref[...] += hg


def _tc_tail(h1, mx, gid2d, ws, wn, bs, we, be):
    B = 2048
    NP = 102400
    h1 = jnp.pad(h1, ((0, NP - N), (0, 0)))
    mx = jnp.pad(mx, ((0, NP - N), (0, 0)))
    gid2d = jnp.pad(gid2d, ((0, 0), (0, NP - N)), constant_values=G + 7)
    grid = NP // B
    return pl.pallas_call(
        _tc_tail_body,
        grid=(grid,),
        in_specs=[
            pl.BlockSpec((B, 32), lambda i: (i, 0)),
            pl.BlockSpec((B, 32), lambda i: (i, 0)),
            pl.BlockSpec((1, B), lambda i: (0, i)),
            pl.BlockSpec((32, 64), lambda i: (0, 0)),
            pl.BlockSpec((32, 64), lambda i: (0, 0)),
            pl.BlockSpec((1, 64), lambda i: (0, 0)),
            pl.BlockSpec((64, 64), lambda i: (0, 0)),
            pl.BlockSpec((1, 64), lambda i: (0, 0)),
        ],
        out_specs=pl.BlockSpec((G, 64), lambda i: (0, 0)),
        out_shape=jax.ShapeDtypeStruct((G, 64), jnp.float32),
    )(h1, mx, gid2d, ws, wn, bs, we, be)


def _top_kernel(hg1_ref, hg2_ref, wt_ref, bt_ref, out_ref):
    z1 = jnp.maximum(
        jnp.dot(hg1_ref[...], wt_ref[...], preferred_element_type=jnp.float32)
        + bt_ref[...], 0.0)
    z2 = jnp.maximum(
        jnp.dot(hg2_ref[...], wt_ref[...], preferred_element_type=jnp.float32)
        + bt_ref[...], 0.0)
    num = jnp.sum(z1 * z2, axis=1)
    n1 = jnp.sqrt(jnp.sum(z1 * z1, axis=1))
    n2 = jnp.sqrt(jnp.sum(z2 * z2, axis=1))
    sim = num / jnp.maximum(n1 * n2, 1e-8)
    out_ref[...] = jnp.clip(jnp.abs(sim), 0.0, 1.0)


def kernel(tokens1, edge_index1, graph_ids1, tokens2, edge_index2, graph_ids2,
           embed, W_gcn, b_gcn, W_pool, b_pool, W_self, W_neigh, b_sage,
           W_enc, b_enc, W_top, b_top):
    i32 = jnp.int32
    tok1 = tokens1.astype(i32)
    tok2 = tokens2.astype(i32)
    src1 = edge_index1[0].astype(i32)
    dst1 = edge_index1[1].astype(i32)
    src2 = edge_index2[0].astype(i32)
    dst2 = edge_index2[1].astype(i32)
    gid1 = graph_ids1.astype(i32).reshape(1, N)
    gid2 = graph_ids2.astype(i32).reshape(1, N)

    h1, h2, do1, di1, do2, di2 = _sc_prep(
        tok1, tok2, embed, src1, dst1, src2, dst2)
    hs1, hs2 = _sc_scale(h1, do1, h2, do2)
    agg1 = _sc_gcn(hs1, src1, dst1, di1)
    agg2 = _sc_gcn(hs2, src2, dst2, di2)

    bg = b_gcn.reshape(1, 32)
    bp = b_pool.reshape(1, 32)
    h1a, hp1 = _tc_h1(agg1, W_gcn, bg, W_pool, bp)
    h1b, hp2 = _tc_h1(agg2, W_gcn, bg, W_pool, bp)

    mx1f, _, _ = _sc_max(hp1, src1, dst1)
    mx2f, _, _ = _sc_max(hp2, src2, dst2)
    mx1 = mx1f.reshape(N, 32)
    mx2 = mx2f.reshape(N, 32)

    bs = b_sage.reshape(1, 64)
    be = b_enc.reshape(1, 64)
    hg1 = _tc_tail(h1a, mx1, gid1, W_self, W_neigh, bs, W_enc, be)
    hg2 = _tc_tail(h1b, mx2, gid2, W_self, W_neigh, bs, W_enc, be)

    sim = pl.pallas_call(
        _top_kernel,
        out_shape=jax.ShapeDtypeStruct((G,), jnp.float32),
    )(hg1, hg2, W_top, b_top.reshape(1, 128))
    return sim
